# baseline (speedup 1.0000x reference)
import jax
import jax.numpy as jnp
from jax import lax
from jax.experimental import pallas as pl
from jax.experimental.pallas import tpu as pltpu
from jax.experimental.pallas import tpu_sc as plsc

NE = 8
TOPK = 2
TM = 512
N_TILES = 8192 // TM + NE
M_PAD = N_TILES * TM


def _gating_body(x_ref, wg_ref, s1_ref, s2_ref, g1_ref, g2_ref, eid_ref,
                 used_ref):
    x = x_ref[...]
    logits = lax.dot_general(x, wg_ref[...], (((1,), (0,)), ((), ())),
                             preferred_element_type=jnp.float32)
    n, e = logits.shape
    iota = lax.broadcasted_iota(jnp.int32, (n, e), 1)
    m1 = jnp.max(logits, axis=1, keepdims=True)
    i1 = jnp.min(jnp.where(logits == m1, iota, e), axis=1, keepdims=True)
    sel1 = iota == i1
    masked = jnp.where(sel1, -jnp.inf, logits)
    m2 = jnp.max(masked, axis=1, keepdims=True)
    i2 = jnp.min(jnp.where(masked == m2, iota, e), axis=1, keepdims=True)
    sel2 = iota == i2
    e2 = jnp.exp(m2 - m1)
    denom = 1.0 + e2
    g1_ref[...] = 1.0 / denom
    g2_ref[...] = e2 / denom

    cnt = sel1.astype(jnp.int32) + sel2.astype(jnp.int32)
    csum = cnt
    row = lax.broadcasted_iota(jnp.int32, (n, e), 0)
    sh = 1
    while sh < n:
        csum = csum + jnp.where(row >= sh, pltpu.roll(csum, sh, 0), 0)
        sh *= 2
    excl = csum - cnt
    rank1 = jnp.sum(jnp.where(sel1, excl, 0), axis=1, keepdims=True)
    rank2 = jnp.sum(jnp.where(sel2, excl, 0), axis=1, keepdims=True)

    counts = jnp.sum(cnt, axis=0, keepdims=True)
    ntile = (counts + (TM - 1)) // TM
    tri = (lax.broadcasted_iota(jnp.int32, (e, e), 0)
           < lax.broadcasted_iota(jnp.int32, (e, e), 1)).astype(jnp.float32)
    toff = lax.dot_general(ntile.astype(jnp.float32), tri,
                           (((1,), (0,)), ((), ())),
                           preferred_element_type=jnp.float32).astype(jnp.int32)
    offs = toff * TM
    s1_ref[...] = jnp.sum(jnp.where(sel1, offs, 0), axis=1,
                          keepdims=True) + rank1
    s2_ref[...] = jnp.sum(jnp.where(sel2, offs, 0), axis=1,
                          keepdims=True) + rank2
    used_ref[...] = jnp.sum(ntile, axis=1, keepdims=True)

    tend = toff + ntile
    t_iota = lax.broadcasted_iota(jnp.int32, (N_TILES, e), 0)
    eid = jnp.sum((t_iota >= tend).astype(jnp.int32), axis=1, keepdims=True)
    eid_ref[...] = jnp.minimum(eid, e - 1)


def _gating(x, w_gate):
    n, d_in = x.shape
    return pl.pallas_call(
        _gating_body,
        in_specs=[pl.BlockSpec((n, d_in), lambda: (0, 0)),
                  pl.BlockSpec((d_in, NE), lambda: (0, 0))],
        out_specs=[pl.BlockSpec((n, 1), lambda: (0, 0)),
                   pl.BlockSpec((n, 1), lambda: (0, 0)),
                   pl.BlockSpec((n, 1), lambda: (0, 0)),
                   pl.BlockSpec((n, 1), lambda: (0, 0)),
                   pl.BlockSpec((N_TILES, 1), lambda: (0, 0)),
                   pl.BlockSpec((1, 1), lambda: (0, 0))],
        out_shape=[jax.ShapeDtypeStruct((n, 1), jnp.int32),
                   jax.ShapeDtypeStruct((n, 1), jnp.int32),
                   jax.ShapeDtypeStruct((n, 1), jnp.float32),
                   jax.ShapeDtypeStruct((n, 1), jnp.float32),
                   jax.ShapeDtypeStruct((N_TILES, 1), jnp.int32),
                   jax.ShapeDtypeStruct((1, 1), jnp.int32)],
    )(x, w_gate)


_SC_MESH = plsc.VectorSubcoreMesh(core_axis_name="c", subcore_axis_name="s")
_NC = 2
_NS = 16
_SCH = 128
_GCH = 48


def _route_body(x_hbm, pos_hbm, tok_hbm, g_hbm, xs_hbm, gsl_hbm, zero_v,
                zerof_v, pos_v, tok_v, gv_v, idx_v, row_v, gout_v, src_sh,
                gsl_sh, sem):
    cid = lax.axis_index("c")
    sid = lax.axis_index("s")
    wid = sid * _NC + cid
    zstripe = M_PAD // _NS
    nch = pos_hbm.shape[1]

    def _z(i, _):
        zero_v[pl.ds(i * 16, 16)] = jnp.zeros((16,), jnp.int32)
        zerof_v[pl.ds(i * 16, 16)] = jnp.zeros((16,), jnp.float32)
        return _
    lax.fori_loop(0, zstripe // 16, _z, 0)
    pltpu.sync_copy(zero_v, src_sh.at[pl.ds(sid * zstripe, zstripe)])
    pltpu.sync_copy(zerof_v, gsl_sh.at[pl.ds(sid * zstripe, zstripe)])
    plsc.subcore_barrier()

    pltpu.sync_copy(pos_hbm.at[sid], pos_v)
    pltpu.sync_copy(tok_hbm.at[sid], tok_v)
    pltpu.sync_copy(g_hbm.at[sid], gv_v)
    for j in range(4):
        pltpu.sync_copy(tok_v.at[j], src_sh.at[pos_v.at[j]])
        pltpu.sync_copy(gv_v.at[j], gsl_sh.at[pos_v.at[j]])
    plsc.subcore_barrier()

    rows_per_w = M_PAD // (_NC * _NS)

    def _chunk(k, _):
        base = wid * rows_per_w + k * _GCH
        pltpu.sync_copy(src_sh.at[pl.ds(base, _GCH)], idx_v)
        pltpu.async_copy(x_hbm.at[idx_v], row_v, sem).wait()
        pltpu.sync_copy(row_v, xs_hbm.at[pl.ds(base, _GCH)])
        return _
    lax.fori_loop(0, rows_per_w // _GCH, _chunk, 0)

    @pl.when(cid == 0)
    def _gout():
        pltpu.sync_copy(gsl_sh.at[pl.ds(sid * zstripe, zstripe)], gout_v)
        pltpu.sync_copy(gout_v, gsl_hbm.at[pl.ds(sid * zstripe, zstripe)])


def _route(x, pos3, tok3, g3):
    n, d_in = x.shape
    k = pl.kernel(
        _route_body,
        out_type=[jax.ShapeDtypeStruct((M_PAD, d_in), jnp.float32),
                  jax.ShapeDtypeStruct((M_PAD,), jnp.float32)],
        mesh=_SC_MESH,
        scratch_types=[
            pltpu.VMEM((M_PAD // _NS,), jnp.int32),
            pltpu.VMEM((M_PAD // _NS,), jnp.float32),
            pltpu.VMEM((4, _SCH), jnp.int32),
            pltpu.VMEM((4, _SCH), jnp.int32),
            pltpu.VMEM((4, _SCH), jnp.float32),
            pltpu.VMEM((_GCH,), jnp.int32),
            pltpu.VMEM((_GCH, d_in), jnp.float32),
            pltpu.VMEM((M_PAD // _NS,), jnp.float32),
            pltpu.VMEM_SHARED((M_PAD,), jnp.int32),
            pltpu.VMEM_SHARED((M_PAD,), jnp.float32),
            pltpu.SemaphoreType.DMA,
        ],
    )
    return k(x, pos3, tok3, g3)


def _gmm_body(eid_ref, used_ref, xs_ref, gs_ref, b_ref, w_ref, y_ref):
    t = pl.program_id(0)

    @pl.when(t < used_ref[0, 0])
    def _compute():
        xs = xs_ref[...] - b_ref[0]
        y = lax.dot_general(xs, w_ref[0], (((1,), (1,)), ((), ())),
                            preferred_element_type=jnp.float32)
        y_ref[...] = y * gs_ref[...]


def _gmm(eid, used, xs, gsl, expert_bias, expert_weight):
    m_pad, d_in = xs.shape
    e, d_out, _ = expert_weight.shape
    return pl.pallas_call(
        _gmm_body,
        grid_spec=pltpu.PrefetchScalarGridSpec(
            num_scalar_prefetch=2,
            grid=(N_TILES,),
            in_specs=[
                pl.BlockSpec((TM, d_in), lambda t, eid, used: (t, 0)),
                pl.BlockSpec((TM, 1), lambda t, eid, used: (t, 0)),
                pl.BlockSpec((1, 1, d_in),
                             lambda t, eid, used: (eid[t, 0], 0, 0)),
                pl.BlockSpec((1, d_out, d_in),
                             lambda t, eid, used: (eid[t, 0], 0, 0)),
            ],
            out_specs=pl.BlockSpec((TM, d_out), lambda t, eid, used: (t, 0)),
        ),
        out_shape=jax.ShapeDtypeStruct((m_pad, d_out), jnp.float32),
        compiler_params=pltpu.CompilerParams(
            dimension_semantics=("arbitrary",)),
    )(eid, used, xs, gsl.reshape(m_pad, 1), expert_bias.reshape(e, 1, d_in),
      expert_weight)


_CCH = 16


def _combine_body(y_hbm, pos_hbm, out_hbm, pos_v, row_v, out_v, sem):
    cid = lax.axis_index("c")
    sid = lax.axis_index("s")
    wid = sid * _NC + cid
    n_tok = out_hbm.shape[0]
    d = out_hbm.shape[1]
    tok_per_w = n_tok // (_NC * _NS)

    pltpu.sync_copy(pos_hbm.at[pl.ds(wid * 2 * tok_per_w, 2 * tok_per_w)],
                    pos_v)

    def _chunk(ci, _):
        pltpu.async_copy(
            y_hbm.at[pos_v.at[pl.ds(ci * 2 * _CCH, 2 * _CCH)]], row_v,
            sem).wait()

        def _tok(i, _):
            def _slice(j, _):
                a = row_v[2 * i, pl.ds(j * 16, 16)]
                b = row_v[2 * i + 1, pl.ds(j * 16, 16)]
                out_v[i, pl.ds(j * 16, 16)] = a + b
                return _
            lax.fori_loop(0, d // 16, _slice, 0)
            return _
        lax.fori_loop(0, _CCH, _tok, 0)
        pltpu.sync_copy(
            out_v, out_hbm.at[pl.ds(wid * tok_per_w + ci * _CCH, _CCH)])
        return _
    lax.fori_loop(0, tok_per_w // _CCH, _chunk, 0)


def _combine(y, pos_flat, n_tok, d_out):
    k = pl.kernel(
        _combine_body,
        out_type=jax.ShapeDtypeStruct((n_tok, d_out), jnp.float32),
        mesh=_SC_MESH,
        scratch_types=[
            pltpu.VMEM((2 * (n_tok // 32),), jnp.int32),
            pltpu.VMEM((2 * _CCH, d_out), jnp.float32),
            pltpu.VMEM((_CCH, d_out), jnp.float32),
            pltpu.SemaphoreType.DMA,
        ],
    )
    return k(y, pos_flat)


def kernel(x, w_gate, expert_bias, expert_weight):
    n_tok, _ = x.shape
    _, d_out, _ = expert_weight.shape
    s1, s2, g1, g2, eid, used = _gating(x, w_gate)
    pos_flat = jnp.concatenate([s1, s2], axis=1).reshape(-1)
    gates_flat = jnp.concatenate([g1, g2], axis=1).reshape(-1)
    pos3 = pos_flat.reshape(_NS, 4, _SCH)
    g3 = gates_flat.reshape(_NS, 4, _SCH)
    tok3 = (jnp.arange(2 * n_tok, dtype=jnp.int32) // 2).reshape(
        _NS, 4, _SCH)
    xs, gsl = _route(x, pos3, tok3, g3)
    y = _gmm(eid, used, xs, gsl, expert_bias, expert_weight)
    return _combine(y, pos_flat, n_tok, d_out)


# speedup vs baseline: 1.0374x; 1.0374x over previous
import jax
import jax.numpy as jnp
from jax import lax
from jax.experimental import pallas as pl
from jax.experimental.pallas import tpu as pltpu
from jax.experimental.pallas import tpu_sc as plsc

NE = 8
TOPK = 2
TM = 512
N_TILES = 8192 // TM + NE
M_PAD = N_TILES * TM


def _gating_body(x_ref, wg_ref, s1_ref, s2_ref, g1_ref, g2_ref, eid_ref,
                 used_ref):
    x = x_ref[...]
    logits = lax.dot_general(x, wg_ref[...], (((1,), (0,)), ((), ())),
                             preferred_element_type=jnp.float32)
    n, e = logits.shape
    iota = lax.broadcasted_iota(jnp.int32, (n, e), 1)
    m1 = jnp.max(logits, axis=1, keepdims=True)
    i1 = jnp.min(jnp.where(logits == m1, iota, e), axis=1, keepdims=True)
    sel1 = iota == i1
    masked = jnp.where(sel1, -jnp.inf, logits)
    m2 = jnp.max(masked, axis=1, keepdims=True)
    i2 = jnp.min(jnp.where(masked == m2, iota, e), axis=1, keepdims=True)
    sel2 = iota == i2
    e2 = jnp.exp(m2 - m1)
    denom = 1.0 + e2
    g1_ref[...] = 1.0 / denom
    g2_ref[...] = e2 / denom

    cnt = sel1.astype(jnp.int32) + sel2.astype(jnp.int32)
    csum = cnt
    row = lax.broadcasted_iota(jnp.int32, (n, e), 0)
    sh = 1
    while sh < n:
        csum = csum + jnp.where(row >= sh, pltpu.roll(csum, sh, 0), 0)
        sh *= 2
    excl = csum - cnt
    rank1 = jnp.sum(jnp.where(sel1, excl, 0), axis=1, keepdims=True)
    rank2 = jnp.sum(jnp.where(sel2, excl, 0), axis=1, keepdims=True)

    counts = jnp.sum(cnt, axis=0, keepdims=True)
    ntile = (counts + (TM - 1)) // TM
    tri = (lax.broadcasted_iota(jnp.int32, (e, e), 0)
           < lax.broadcasted_iota(jnp.int32, (e, e), 1)).astype(jnp.float32)
    toff = lax.dot_general(ntile.astype(jnp.float32), tri,
                           (((1,), (0,)), ((), ())),
                           preferred_element_type=jnp.float32).astype(jnp.int32)
    offs = toff * TM
    s1_ref[...] = jnp.sum(jnp.where(sel1, offs, 0), axis=1,
                          keepdims=True) + rank1
    s2_ref[...] = jnp.sum(jnp.where(sel2, offs, 0), axis=1,
                          keepdims=True) + rank2
    used_ref[...] = jnp.sum(ntile, axis=1, keepdims=True)

    tend = toff + ntile
    t_iota = lax.broadcasted_iota(jnp.int32, (N_TILES, e), 0)
    eid = jnp.sum((t_iota >= tend).astype(jnp.int32), axis=1, keepdims=True)
    eid_ref[...] = jnp.minimum(eid, e - 1)


def _gating(x, w_gate):
    n, d_in = x.shape
    return pl.pallas_call(
        _gating_body,
        in_specs=[pl.BlockSpec((n, d_in), lambda: (0, 0)),
                  pl.BlockSpec((d_in, NE), lambda: (0, 0))],
        out_specs=[pl.BlockSpec((n, 1), lambda: (0, 0)),
                   pl.BlockSpec((n, 1), lambda: (0, 0)),
                   pl.BlockSpec((n, 1), lambda: (0, 0)),
                   pl.BlockSpec((n, 1), lambda: (0, 0)),
                   pl.BlockSpec((N_TILES, 1), lambda: (0, 0)),
                   pl.BlockSpec((1, 1), lambda: (0, 0))],
        out_shape=[jax.ShapeDtypeStruct((n, 1), jnp.int32),
                   jax.ShapeDtypeStruct((n, 1), jnp.int32),
                   jax.ShapeDtypeStruct((n, 1), jnp.float32),
                   jax.ShapeDtypeStruct((n, 1), jnp.float32),
                   jax.ShapeDtypeStruct((N_TILES, 1), jnp.int32),
                   jax.ShapeDtypeStruct((1, 1), jnp.int32)],
    )(x, w_gate)


_SC_MESH = plsc.VectorSubcoreMesh(core_axis_name="c", subcore_axis_name="s")
_NC = 2
_NS = 16
_SCH = 128
_GCH = 48


def _route_body(x_hbm, pos_hbm, tok_hbm, g_hbm, xs_hbm, gsl_hbm, zero_v,
                zerof_v, pos_v, tok_v, gv_v, idx_v, row0_v, row1_v, gout_v,
                src_sh, gsl_sh, sem, gs0, gs1, ws0, ws1):
    cid = lax.axis_index("c")
    sid = lax.axis_index("s")
    wid = sid * _NC + cid
    zstripe = M_PAD // _NS

    def _z(i, _):
        zero_v[pl.ds(i * 16, 16)] = jnp.zeros((16,), jnp.int32)
        zerof_v[pl.ds(i * 16, 16)] = jnp.zeros((16,), jnp.float32)
        return _
    lax.fori_loop(0, zstripe // 16, _z, 0)
    pltpu.sync_copy(zero_v, src_sh.at[pl.ds(sid * zstripe, zstripe)])
    pltpu.sync_copy(zerof_v, gsl_sh.at[pl.ds(sid * zstripe, zstripe)])
    plsc.subcore_barrier()

    pltpu.sync_copy(pos_hbm.at[sid], pos_v)
    pltpu.sync_copy(tok_hbm.at[sid], tok_v)
    pltpu.sync_copy(g_hbm.at[sid], gv_v)
    descs = []
    for j in range(4):
        descs.append(
            pltpu.async_copy(tok_v.at[j], src_sh.at[pos_v.at[j]], gs0))
        descs.append(
            pltpu.async_copy(gv_v.at[j], gsl_sh.at[pos_v.at[j]], gs1))
    for d_ in descs:
        d_.wait()
    plsc.subcore_barrier()

    rows_per_w = M_PAD // (_NC * _NS)
    nch = rows_per_w // _GCH
    base0 = wid * rows_per_w
    pltpu.sync_copy(src_sh.at[pl.ds(base0, rows_per_w)], idx_v)

    bufs = (row0_v, row1_v)
    gsems = (gs0, gs1)
    wsems = (ws0, ws1)
    gd = [None] * nch
    wd = [None] * nch
    gd[0] = pltpu.async_copy(
        x_hbm.at[idx_v.at[pl.ds(0, _GCH)]], bufs[0], gsems[0])
    for k in range(nch):
        b = k % 2
        if k + 1 < nch:
            if k >= 1:
                wd[k - 1].wait()
            gd[k + 1] = pltpu.async_copy(
                x_hbm.at[idx_v.at[pl.ds((k + 1) * _GCH, _GCH)]],
                bufs[(k + 1) % 2], gsems[(k + 1) % 2])
        gd[k].wait()
        wd[k] = pltpu.async_copy(
            bufs[b], xs_hbm.at[pl.ds(base0 + k * _GCH, _GCH)], wsems[b])
    wd[nch - 1].wait()

    @pl.when(cid == 0)
    def _gout():
        pltpu.sync_copy(gsl_sh.at[pl.ds(sid * zstripe, zstripe)], gout_v)
        pltpu.sync_copy(gout_v, gsl_hbm.at[pl.ds(sid * zstripe, zstripe)])


def _route(x, pos3, tok3, g3):
    n, d_in = x.shape
    k = pl.kernel(
        _route_body,
        out_type=[jax.ShapeDtypeStruct((M_PAD, d_in), jnp.float32),
                  jax.ShapeDtypeStruct((M_PAD,), jnp.float32)],
        mesh=_SC_MESH,
        scratch_types=[
            pltpu.VMEM((M_PAD // _NS,), jnp.int32),
            pltpu.VMEM((M_PAD // _NS,), jnp.float32),
            pltpu.VMEM((4, _SCH), jnp.int32),
            pltpu.VMEM((4, _SCH), jnp.int32),
            pltpu.VMEM((4, _SCH), jnp.float32),
            pltpu.VMEM((M_PAD // (_NC * _NS),), jnp.int32),
            pltpu.VMEM((_GCH, d_in), jnp.float32),
            pltpu.VMEM((_GCH, d_in), jnp.float32),
            pltpu.VMEM((M_PAD // _NS,), jnp.float32),
            pltpu.VMEM_SHARED((M_PAD,), jnp.int32),
            pltpu.VMEM_SHARED((M_PAD,), jnp.float32),
            pltpu.SemaphoreType.DMA,
            pltpu.SemaphoreType.DMA,
            pltpu.SemaphoreType.DMA,
            pltpu.SemaphoreType.DMA,
            pltpu.SemaphoreType.DMA,
        ],
    )
    return k(x, pos3, tok3, g3)


def _gmm_body(eid_ref, used_ref, xs_ref, gs_ref, b_ref, w_ref, y_ref):
    t = pl.program_id(0)

    @pl.when(t < used_ref[0, 0])
    def _compute():
        xs = xs_ref[...] - b_ref[0]
        y = lax.dot_general(xs, w_ref[0], (((1,), (1,)), ((), ())),
                            preferred_element_type=jnp.float32)
        y_ref[...] = y * gs_ref[...]


def _gmm(eid, used, xs, gsl, expert_bias, expert_weight):
    m_pad, d_in = xs.shape
    e, d_out, _ = expert_weight.shape
    return pl.pallas_call(
        _gmm_body,
        grid_spec=pltpu.PrefetchScalarGridSpec(
            num_scalar_prefetch=2,
            grid=(N_TILES,),
            in_specs=[
                pl.BlockSpec((TM, d_in), lambda t, eid, used: (t, 0)),
                pl.BlockSpec((TM, 1), lambda t, eid, used: (t, 0)),
                pl.BlockSpec((1, 1, d_in),
                             lambda t, eid, used: (eid[t, 0], 0, 0)),
                pl.BlockSpec((1, d_out, d_in),
                             lambda t, eid, used: (eid[t, 0], 0, 0)),
            ],
            out_specs=pl.BlockSpec((TM, d_out), lambda t, eid, used: (t, 0)),
        ),
        out_shape=jax.ShapeDtypeStruct((m_pad, d_out), jnp.float32),
        compiler_params=pltpu.CompilerParams(
            dimension_semantics=("arbitrary",)),
    )(eid, used, xs, gsl.reshape(m_pad, 1), expert_bias.reshape(e, 1, d_in),
      expert_weight)


_CCH = 16


def _combine_body(y_hbm, pos_hbm, out_hbm, pos_v, row0_v, row1_v, out0_v,
                  out1_v, gs0, gs1, ws0, ws1):
    cid = lax.axis_index("c")
    sid = lax.axis_index("s")
    wid = sid * _NC + cid
    n_tok = out_hbm.shape[0]
    d = out_hbm.shape[1]
    tok_per_w = n_tok // (_NC * _NS)
    nch = tok_per_w // _CCH

    pltpu.sync_copy(pos_hbm.at[pl.ds(wid * 2 * tok_per_w, 2 * tok_per_w)],
                    pos_v)

    rbufs = (row0_v, row1_v)
    obufs = (out0_v, out1_v)
    gsems = (gs0, gs1)
    wsems = (ws0, ws1)
    gd = [None] * nch
    wd = [None] * nch
    gd[0] = pltpu.async_copy(
        y_hbm.at[pos_v.at[pl.ds(0, 2 * _CCH)]], rbufs[0], gsems[0])
    for ci in range(nch):
        b = ci % 2
        if ci + 1 < nch:
            gd[ci + 1] = pltpu.async_copy(
                y_hbm.at[pos_v.at[pl.ds((ci + 1) * 2 * _CCH, 2 * _CCH)]],
                rbufs[(ci + 1) % 2], gsems[(ci + 1) % 2])
        gd[ci].wait()
        if ci >= 2:
            wd[ci - 2].wait()
        row_v = rbufs[b]
        out_v = obufs[b]

        def _tok(i, _):
            for j in range(d // 16):
                a = row_v[2 * i, pl.ds(j * 16, 16)]
                bb = row_v[2 * i + 1, pl.ds(j * 16, 16)]
                out_v[i, pl.ds(j * 16, 16)] = a + bb
            return _
        lax.fori_loop(0, _CCH, _tok, 0)
        wd[ci] = pltpu.async_copy(
            out_v, out_hbm.at[pl.ds(wid * tok_per_w + ci * _CCH, _CCH)],
            wsems[b])
    wd[nch - 1].wait()
    if nch >= 2:
        wd[nch - 2].wait()


def _combine(y, pos_flat, n_tok, d_out):
    k = pl.kernel(
        _combine_body,
        out_type=jax.ShapeDtypeStruct((n_tok, d_out), jnp.float32),
        mesh=_SC_MESH,
        scratch_types=[
            pltpu.VMEM((2 * (n_tok // 32),), jnp.int32),
            pltpu.VMEM((2 * _CCH, d_out), jnp.float32),
            pltpu.VMEM((2 * _CCH, d_out), jnp.float32),
            pltpu.VMEM((_CCH, d_out), jnp.float32),
            pltpu.VMEM((_CCH, d_out), jnp.float32),
            pltpu.SemaphoreType.DMA,
            pltpu.SemaphoreType.DMA,
            pltpu.SemaphoreType.DMA,
            pltpu.SemaphoreType.DMA,
        ],
    )
    return k(y, pos_flat)


def kernel(x, w_gate, expert_bias, expert_weight):
    n_tok, _ = x.shape
    _, d_out, _ = expert_weight.shape
    s1, s2, g1, g2, eid, used = _gating(x, w_gate)
    pos_flat = jnp.concatenate([s1, s2], axis=1).reshape(-1)
    gates_flat = jnp.concatenate([g1, g2], axis=1).reshape(-1)
    pos3 = pos_flat.reshape(_NS, 4, _SCH)
    g3 = gates_flat.reshape(_NS, 4, _SCH)
    tok3 = (jnp.arange(2 * n_tok, dtype=jnp.int32) // 2).reshape(
        _NS, 4, _SCH)
    xs, gsl = _route(x, pos3, tok3, g3)
    y = _gmm(eid, used, xs, gsl, expert_bias, expert_weight)
    return _combine(y, pos_flat, n_tok, d_out)


# speedup vs baseline: 3.3437x; 3.2231x over previous
import functools

import jax
import jax.numpy as jnp
from jax.experimental import pallas as pl
from jax.experimental.pallas import tpu as pltpu

E = 8
K = 2


def _moe_body(x_ref, wg_ref, b_ref, w_ref, out_ref, gates_ref, *, n_tiles):
    n_i = pl.program_id(0)
    e_i = pl.program_id(1)

    @pl.when(jnp.logical_and(n_i == 0, e_i == 0))
    def _compute_gates():
        x = x_ref[...]
        logits = jax.lax.dot_general(
            x, wg_ref[...], (((1,), (0,)), ((), ())),
            preferred_element_type=jnp.float32)
        n, e = logits.shape
        iota = jax.lax.broadcasted_iota(jnp.int32, (n, e), 1)
        m1 = jnp.max(logits, axis=1, keepdims=True)
        i1 = jnp.min(jnp.where(logits == m1, iota, e), axis=1, keepdims=True)
        sel1 = iota == i1
        masked = jnp.where(sel1, -jnp.inf, logits)
        m2 = jnp.max(masked, axis=1, keepdims=True)
        i2 = jnp.min(jnp.where(masked == m2, iota, e), axis=1, keepdims=True)
        sel2 = iota == i2
        e2 = jnp.exp(m2 - m1)
        denom = 1.0 + e2
        gates_ref[...] = jnp.where(sel1, 1.0 / denom, 0.0) + jnp.where(
            sel2, e2 / denom, 0.0)

    w = w_ref[0]
    b = b_ref[...]
    xw = jax.lax.dot_general(
        x_ref[...], w, (((1,), (1,)), ((), ())),
        preferred_element_type=jnp.float32)
    row_iota = jax.lax.broadcasted_iota(jnp.int32, b.shape, 0)
    be = jnp.sum(jnp.where(row_iota == e_i, b, 0.0), axis=0, keepdims=True)
    bc = jax.lax.dot_general(
        be, w, (((1,), (1,)), ((), ())), preferred_element_type=jnp.float32)
    gates = gates_ref[...]
    col_iota = jax.lax.broadcasted_iota(jnp.int32, gates.shape, 1)
    g = jnp.sum(jnp.where(col_iota == e_i, gates, 0.0), axis=1, keepdims=True)
    contrib = g * (xw - bc)

    @pl.when(e_i == 0)
    def _init():
        out_ref[...] = contrib

    @pl.when(e_i != 0)
    def _acc():
        out_ref[...] = out_ref[...] + contrib


def kernel(x, w_gate, expert_bias, expert_weight):
    n_tok, d_in = x.shape
    e, d_out, _ = expert_weight.shape
    tn = 512
    n_tiles = d_out // tn
    body = functools.partial(_moe_body, n_tiles=n_tiles)
    return pl.pallas_call(
        body,
        grid=(n_tiles, e),
        in_specs=[
            pl.BlockSpec((n_tok, d_in), lambda n, ei: (0, 0)),
            pl.BlockSpec((d_in, e), lambda n, ei: (0, 0)),
            pl.BlockSpec((e, d_in), lambda n, ei: (0, 0)),
            pl.BlockSpec((1, tn, d_in), lambda n, ei: (ei, n, 0)),
        ],
        out_specs=pl.BlockSpec((n_tok, tn), lambda n, ei: (0, n)),
        out_shape=jax.ShapeDtypeStruct((n_tok, d_out), jnp.float32),
        scratch_shapes=[pltpu.VMEM((n_tok, e), jnp.float32)],
        compiler_params=pltpu.CompilerParams(
            dimension_semantics=("arbitrary", "arbitrary")),
    )(x, w_gate, expert_bias, expert_weight)


# speedup vs baseline: 3.5993x; 1.0764x over previous
import functools

import jax
import jax.numpy as jnp
from jax.experimental import pallas as pl
from jax.experimental.pallas import tpu as pltpu

E = 8
K = 2


def _moe_body(x_ref, wg_ref, b_ref, w_ref, out_ref, gates_ref, bnz_ref,
              bc_ref, *, n_tiles):
    n_i = pl.program_id(0)
    e_i = pl.program_id(1)

    @pl.when(jnp.logical_and(n_i == 0, e_i == 0))
    def _compute_gates():
        bnz_ref[0, 0] = (jnp.max(jnp.abs(b_ref[...])) > 0.0).astype(jnp.int32)
        x = x_ref[...]
        logits = jax.lax.dot_general(
            x, wg_ref[...], (((1,), (0,)), ((), ())),
            preferred_element_type=jnp.float32)
        n, e = logits.shape
        iota = jax.lax.broadcasted_iota(jnp.int32, (n, e), 1)
        m1 = jnp.max(logits, axis=1, keepdims=True)
        i1 = jnp.min(jnp.where(logits == m1, iota, e), axis=1, keepdims=True)
        sel1 = iota == i1
        masked = jnp.where(sel1, -jnp.inf, logits)
        m2 = jnp.max(masked, axis=1, keepdims=True)
        i2 = jnp.min(jnp.where(masked == m2, iota, e), axis=1, keepdims=True)
        sel2 = iota == i2
        e2 = jnp.exp(m2 - m1)
        denom = 1.0 + e2
        gates_ref[...] = jnp.where(sel1, 1.0 / denom, 0.0) + jnp.where(
            sel2, e2 / denom, 0.0)

    w = w_ref[0]
    xw = jax.lax.dot_general(
        x_ref[...], w, (((1,), (1,)), ((), ())),
        preferred_element_type=jnp.float32)

    @pl.when(bnz_ref[0, 0] == 0)
    def _bc_zero():
        bc_ref[...] = jnp.zeros_like(bc_ref)

    @pl.when(bnz_ref[0, 0] != 0)
    def _bc_full():
        b = b_ref[...]
        row_iota = jax.lax.broadcasted_iota(jnp.int32, b.shape, 0)
        be = jnp.sum(jnp.where(row_iota == e_i, b, 0.0), axis=0,
                     keepdims=True)
        bc_ref[...] = jax.lax.dot_general(
            be, w, (((1,), (1,)), ((), ())),
            preferred_element_type=jnp.float32)

    gates = gates_ref[...]
    col_iota = jax.lax.broadcasted_iota(jnp.int32, gates.shape, 1)
    g = jnp.sum(jnp.where(col_iota == e_i, gates, 0.0), axis=1, keepdims=True)
    contrib = g * (xw - bc_ref[...])

    @pl.when(e_i == 0)
    def _init():
        out_ref[...] = contrib

    @pl.when(e_i != 0)
    def _acc():
        out_ref[...] = out_ref[...] + contrib


def kernel(x, w_gate, expert_bias, expert_weight):
    n_tok, d_in = x.shape
    e, d_out, _ = expert_weight.shape
    tn = 512
    n_tiles = d_out // tn
    body = functools.partial(_moe_body, n_tiles=n_tiles)
    return pl.pallas_call(
        body,
        grid=(n_tiles, e),
        in_specs=[
            pl.BlockSpec((n_tok, d_in), lambda n, ei: (0, 0)),
            pl.BlockSpec((d_in, e), lambda n, ei: (0, 0)),
            pl.BlockSpec((e, d_in), lambda n, ei: (0, 0)),
            pl.BlockSpec((1, tn, d_in), lambda n, ei: (ei, n, 0)),
        ],
        out_specs=pl.BlockSpec((n_tok, tn), lambda n, ei: (0, n)),
        out_shape=jax.ShapeDtypeStruct((n_tok, d_out), jnp.float32),
        scratch_shapes=[pltpu.VMEM((n_tok, e), jnp.float32),
                        pltpu.SMEM((1, 1), jnp.int32),
                        pltpu.VMEM((1, tn), jnp.float32)],
        compiler_params=pltpu.CompilerParams(
            dimension_semantics=("arbitrary", "arbitrary")),
    )(x, w_gate, expert_bias, expert_weight)
